# baseline (speedup 1.0000x reference)
import functools

import jax
import jax.numpy as jnp
from jax import lax
from jax.experimental import pallas as pl
from jax.experimental.pallas import tpu as pltpu
from jax.experimental.pallas import tpu_sc as plsc

TM = 1024
CB = 8192

_NC = 2
_NS = 16
_NW = _NC * _NS
_IDX_CHUNK = 128


def _argmin_body(x2_ref, flat_ref, embed_ref, e2_ref, colf_ref, codes_ref):
    dot2 = lax.dot_general(
        flat_ref[...] * -2.0, embed_ref[...], (((1,), (1,)), ((), ()))
    )
    d = (x2_ref[...] + e2_ref[...]) + dot2
    bmin = jnp.min(d, axis=1, keepdims=True)
    bidx = jnp.min(
        jnp.where(d == bmin, colf_ref[...], jnp.inf), axis=1, keepdims=True
    )
    codes_ref[...] = bidx.astype(jnp.int32)


def _codes_call(x2, flat, embed, e2, colf):
    n, c = flat.shape
    v = embed.shape[0]
    assert v == CB
    return pl.pallas_call(
        _argmin_body,
        grid=(n // TM,),
        in_specs=[
            pl.BlockSpec((TM, 1), lambda t: (t, 0)),
            pl.BlockSpec((TM, c), lambda t: (t, 0)),
            pl.BlockSpec((CB, c), lambda t: (0, 0)),
            pl.BlockSpec((1, CB), lambda t: (0, 0)),
            pl.BlockSpec((1, CB), lambda t: (0, 0)),
        ],
        out_specs=pl.BlockSpec((TM, 1), lambda t: (t, 0)),
        out_shape=jax.ShapeDtypeStruct((n, 1), jnp.int32),
        compiler_params=pltpu.CompilerParams(
            dimension_semantics=("arbitrary",)
        ),
    )(x2, flat, embed, e2, colf)


def _make_sc_gather(v, d, b):
    b_per_w = b // _NW
    chunks = b_per_w // _IDX_CHUNK
    mesh = plsc.VectorSubcoreMesh(core_axis_name="c", subcore_axis_name="s")

    @functools.partial(
        pl.kernel,
        mesh=mesh,
        out_type=jax.ShapeDtypeStruct((_NW, chunks, _IDX_CHUNK, d), jnp.float32),
        scratch_types=[
            pltpu.VMEM((chunks, _IDX_CHUNK), jnp.int32),
            pltpu.VMEM((chunks, _IDX_CHUNK, d), jnp.float32),
            pltpu.SemaphoreType.DMA,
        ],
    )
    def gk(table_hbm, idx_hbm, out_hbm, idx_v, rows_v, sem):
        wid = lax.axis_index("s") * _NC + lax.axis_index("c")
        pltpu.sync_copy(idx_hbm.at[pl.ds(wid * chunks, chunks)], idx_v)
        copies = []
        for j in range(chunks):
            copies.append(
                pltpu.async_copy(table_hbm.at[idx_v.at[j]], rows_v.at[j], sem)
            )
        for cp in copies:
            cp.wait()
        pltpu.sync_copy(rows_v, out_hbm.at[wid])

    return gk


def kernel(z, embed):
    b, c, h, w = z.shape
    v = embed.shape[0]
    flat = jnp.transpose(z, (0, 2, 3, 1)).reshape(-1, c)
    n = flat.shape[0]
    x2 = jnp.sum(flat * flat, axis=1, keepdims=True)
    e2 = jnp.sum(embed * embed, axis=1, keepdims=True).T
    colf = jnp.arange(v, dtype=jnp.float32).reshape(1, v)

    codes = _codes_call(x2, flat, embed, e2, colf).reshape(n)

    idx2d = codes.reshape(n // _IDX_CHUNK, _IDX_CHUNK)
    zq_rows = _make_sc_gather(v, c, n)(embed, idx2d)
    z_q = jnp.transpose(zq_rows.reshape(b, h, w, c), (0, 3, 1, 2))
    return (z_q, codes.reshape(b, h, w))
